# baseline (speedup 1.0000x reference)
import functools

import jax
import jax.numpy as jnp
from jax import lax
from jax.experimental import pallas as pl
from jax.experimental.pallas import tpu as pltpu
from jax.experimental.pallas import tpu_sc as plsc

_B, _T, _D = 4, 512, 256
_P, _K = 12, 100
_INVTEMP = 10.0
_NR = _B * _T
_NROWS = _P * _NR
_LANES = 16
_KTOT = _K + 1
_NG = _NROWS // _LANES
_NCORES, _NSUB = 2, 16
_NW = _NCORES * _NSUB
_GPW = _NG // _NW


_KPAD = 128


def _build_gather_cols():
    import numpy as np

    r_in_step = np.arange(_NR, dtype=np.int64)
    steps = []
    for s in range(_P):
        step = s + 1
        t2 = _T - step
        kstep = jax.random.fold_in(jax.random.key(42), step)
        neg = np.asarray(
            jax.random.randint(kstep, (_B, t2, _K), 0, _B * t2),
            dtype=np.int64)
        negc = (neg // t2) * _T + (neg % t2) + step
        negc = np.pad(negc, ((0, 0), (0, _T - t2), (1, _KPAD - 1 - _K)))
        posc = np.minimum(r_in_step + step, _NR - 1).reshape(_B, _T)
        posc = np.pad(posc[:, :, None], ((0, 0), (0, 0), (0, _KPAD - 1)))
        col = negc + posc
        row = r_in_step.reshape(_B, _T, 1)
        steps.append((s * 16 + (col >> 7)) * (_NR * 128) + row * 128
                     + (col & 127))
    arr = np.stack(steps).reshape(_NG, _LANES, _KPAD)
    return np.ascontiguousarray(
        arr.transpose(0, 2, 1)).reshape(-1).astype(np.int32)


def _idx_const_np():
    try:
        cpu = jax.local_devices(backend="cpu")[0]
        with jax.default_device(cpu):
            return _build_gather_cols()
    except RuntimeError:
        return _build_gather_cols()


_IDX_CONST = _idx_const_np()


def _sim_body(ctx_ref, tgt_ref, w_ref, b_ref, s_ref):
    tgt = tgt_ref[...]
    tn = tgt / jnp.maximum(
        jnp.sqrt(jnp.sum(tgt * tgt, axis=1, keepdims=True)), 1e-12)
    q = lax.dot_general(ctx_ref[...], w_ref[0], (((1,), (1,)), ((), ())),
                        preferred_element_type=jnp.float32)
    q = q + b_ref[0, 0][None, :]
    qn = q / jnp.maximum(
        jnp.sqrt(jnp.sum(q * q, axis=1, keepdims=True)), 1e-12)
    for cb in range(_NR // 128):
        s_ref[pl.ds(cb * _NR, _NR), :] = lax.dot_general(
            qn, tn[cb * 128:(cb + 1) * 128, :], (((1,), (1,)), ((), ())),
            preferred_element_type=jnp.float32)


_sim = pl.pallas_call(
    _sim_body,
    grid=(_P,),
    in_specs=[
        pl.BlockSpec((_NR, _D), lambda s: (0, 0)),
        pl.BlockSpec((_NR, _D), lambda s: (0, 0)),
        pl.BlockSpec((1, _D, _D), lambda s: (s, 0, 0)),
        pl.BlockSpec((1, 1, _D), lambda s: (s, 0, 0)),
    ],
    out_specs=pl.BlockSpec(((_NR // 128) * _NR, 128), lambda s: (s, 0)),
    out_shape=jax.ShapeDtypeStruct((_P * (_NR // 128) * _NR, 128),
                                   jnp.float32),
)


_NCB = _NR // 128
_CHUNK = _LANES * 128


_IDXW = _KPAD * _LANES
_NIB = _IDXW // 128


def _sc_gather_body(s_hbm, idx_hbm, sum_hbm, pos_hbm,
                    gbuf0, gbuf1, idxbuf0, idxbuf1, vsum, vpos,
                    semg0, semg1, semi0, semi1):
    cid = lax.axis_index("c")
    sid = lax.axis_index("s")
    wid = sid * _NCORES + cid
    base = wid * _GPW

    def issue_idx(gg, idxbuf, sem):
        ggc = jnp.minimum(gg, _NG - 1)
        pltpu.async_copy(idx_hbm.at[pl.ds(ggc * _IDXW, _IDXW)], idxbuf, sem)

    def drain_idx(idxbuf, sem):
        pltpu.make_async_copy(idx_hbm.at[pl.ds(0, _IDXW)], idxbuf,
                              sem).wait()

    def issue_gather(idxbuf, gbuf, sem):
        for j in range(_NIB):
            pltpu.async_copy(
                s_hbm.at[idxbuf.at[pl.ds(j * 128, 128)]],
                gbuf.at[pl.ds(j * 128, 128)],
                sem)

    def drain_gather(gbuf, sem):
        for j in range(_NIB):
            pltpu.make_async_copy(
                s_hbm.at[pl.ds(0, 128)],
                gbuf.at[pl.ds(j * 128, 128)],
                sem).wait()

    def compute(gg, gbuf):
        v = gbuf[pl.ds(0, _LANES)]
        vpos[:] = v
        acc = jnp.exp(v * _INVTEMP)
        for k in range(1, _KTOT):
            acc = acc + jnp.exp(gbuf[pl.ds(k * _LANES, _LANES)] * _INVTEMP)
        vsum[:] = acc
        grow = gg * _LANES
        pltpu.sync_copy(vsum, sum_hbm.at[pl.ds(grow, _LANES)])
        pltpu.sync_copy(vpos, pos_hbm.at[pl.ds(grow, _LANES)])

    gb = (gbuf0, gbuf1)
    ib = (idxbuf0, idxbuf1)
    sg = (semg0, semg1)
    si = (semi0, semi1)

    issue_idx(base, ib[0], si[0])
    drain_idx(ib[0], si[0])
    issue_gather(ib[0], gb[0], sg[0])
    issue_idx(base + 1, ib[1], si[1])

    def body(j, carry):
        for par in range(2):
            gg = base + 2 * j + par
            o = 1 - par
            drain_gather(gb[par], sg[par])
            drain_idx(ib[o], si[o])
            issue_gather(ib[o], gb[o], sg[o])
            issue_idx(gg + 2, ib[par], si[par])
            compute(gg, gb[par])
        return carry

    lax.fori_loop(0, _GPW // 2, body, 0)
    drain_gather(gb[0], sg[0])
    drain_idx(ib[1], si[1])


@functools.cache
def _get_sc_gather():
    return pl.kernel(
        _sc_gather_body,
        mesh=plsc.VectorSubcoreMesh(core_axis_name="c", subcore_axis_name="s"),
        compiler_params=pltpu.CompilerParams(needs_layout_passes=False),
        out_type=[jax.ShapeDtypeStruct((_NROWS,), jnp.float32),
                  jax.ShapeDtypeStruct((_NROWS,), jnp.float32)],
        scratch_types=[
            pltpu.VMEM((_IDXW,), jnp.float32),
            pltpu.VMEM((_IDXW,), jnp.float32),
            pltpu.VMEM((_IDXW,), jnp.int32),
            pltpu.VMEM((_IDXW,), jnp.int32),
            pltpu.VMEM((_LANES,), jnp.float32),
            pltpu.VMEM((_LANES,), jnp.float32),
            pltpu.SemaphoreType.DMA,
            pltpu.SemaphoreType.DMA,
            pltpu.SemaphoreType.DMA,
            pltpu.SemaphoreType.DMA,
        ],
    )


def _loss_body(sum_ref, pos_ref, out_ref):
    se = sum_ref[...]
    ps = pos_ref[...]
    srow = lax.broadcasted_iota(jnp.int32, (_P, _NR), 0)
    rcol = lax.broadcasted_iota(jnp.int32, (_P, _NR), 1)
    t2 = (_T - 1) - srow
    valid = (rcol % _T) < t2
    wgt = jnp.where(valid, 1.0, 0.0) / (_P * _B * t2.astype(jnp.float32))
    out_ref[...] = jnp.sum((jnp.log(se) - ps * _INVTEMP) * wgt).reshape(1, 1)


_loss = pl.pallas_call(
    _loss_body,
    out_shape=jax.ShapeDtypeStruct((1, 1), jnp.float32),
)


def kernel(context, targets, W, b):
    ctx2 = context.reshape(_NR, _D)
    tgt2 = targets.reshape(_NR, _D)
    b3 = b.reshape(_P, 1, _D)
    idx = jnp.asarray(_IDX_CONST)
    sim = _sim(ctx2, tgt2, W, b3)
    se, ps = _get_sc_gather()(sim.reshape(-1), idx)
    return _loss(se.reshape(_P, _NR), ps.reshape(_P, _NR)).reshape(1)


# speedup vs baseline: 1.2151x; 1.2151x over previous
import functools

import jax
import jax.numpy as jnp
from jax import lax
from jax.experimental import pallas as pl
from jax.experimental.pallas import tpu as pltpu
from jax.experimental.pallas import tpu_sc as plsc

_B, _T, _D = 4, 512, 256
_P, _K = 12, 100
_INVTEMP = 10.0
_NR = _B * _T
_NROWS = _P * _NR
_LANES = 16
_KTOT = _K + 1
_NG = _NROWS // _LANES
_NCORES, _NSUB = 2, 16
_NW = _NCORES * _NSUB
_GPW = _NG // _NW


_KPAD = 128


def _build_gather_cols():
    import numpy as np

    lane = (np.arange(_T, dtype=np.int64) % _LANES)

    def remap(col, lane_b):
        return ((col >> 7) * (_LANES * 128) + lane_b * 128
                + (col & 127)).astype(np.int32)

    r_in_step = np.arange(_NR, dtype=np.int64)
    steps = []
    for s in range(_P):
        step = s + 1
        t2 = _T - step
        kstep = jax.random.fold_in(jax.random.key(42), step)
        neg = np.asarray(
            jax.random.randint(kstep, (_B, t2, _K), 0, _B * t2),
            dtype=np.int64)
        negc = (neg // t2) * _T + (neg % t2) + step
        negc = remap(negc, lane[:t2].reshape(1, t2, 1))
        negc = np.pad(negc, ((0, 0), (0, _T - t2), (1, _KPAD - 1 - _K)))
        posc = np.minimum(r_in_step + step, _NR - 1).reshape(_B, _T)
        posc = remap(posc, lane.reshape(1, _T))
        posc = np.pad(posc[:, :, None], ((0, 0), (0, 0), (0, _KPAD - 1)))
        steps.append(negc + posc)
    arr = np.stack(steps).reshape(_NG, _LANES, _KPAD)
    return np.ascontiguousarray(
        arr.transpose(0, 2, 1)).reshape(-1).astype(np.int32)


def _idx_const_np():
    try:
        cpu = jax.local_devices(backend="cpu")[0]
        with jax.default_device(cpu):
            return _build_gather_cols()
    except RuntimeError:
        return _build_gather_cols()


_IDX_CONST = _idx_const_np()


def _sim_body(ctx_ref, tgt_ref, w_ref, b_ref, s_ref):
    tgt = tgt_ref[...]
    tn = tgt / jnp.maximum(
        jnp.sqrt(jnp.sum(tgt * tgt, axis=1, keepdims=True)), 1e-12)
    q = lax.dot_general(ctx_ref[...], w_ref[0], (((1,), (1,)), ((), ())),
                        preferred_element_type=jnp.float32)
    q = q + b_ref[0, 0][None, :]
    qn = q / jnp.maximum(
        jnp.sqrt(jnp.sum(q * q, axis=1, keepdims=True)), 1e-12)
    for cb in range(_NR // 128):
        s_ref[pl.ds(cb * _NR, _NR), :] = lax.dot_general(
            qn, tn[cb * 128:(cb + 1) * 128, :], (((1,), (1,)), ((), ())),
            preferred_element_type=jnp.float32)


_sim = pl.pallas_call(
    _sim_body,
    grid=(_P,),
    in_specs=[
        pl.BlockSpec((_NR, _D), lambda s: (0, 0)),
        pl.BlockSpec((_NR, _D), lambda s: (0, 0)),
        pl.BlockSpec((1, _D, _D), lambda s: (s, 0, 0)),
        pl.BlockSpec((1, 1, _D), lambda s: (s, 0, 0)),
    ],
    out_specs=pl.BlockSpec(((_NR // 128) * _NR, 128), lambda s: (s, 0)),
    out_shape=jax.ShapeDtypeStruct((_P * (_NR // 128) * _NR, 128),
                                   jnp.float32),
)


_NCB = _NR // 128
_CHUNK = _LANES * 128


_IDXW = _KPAD * _LANES


def _sc_gather_body(s_hbm, idx_hbm, sum_hbm, pos_hbm,
                    rowbuf0, rowbuf1, idxbuf0, idxbuf1, vsum, vpos,
                    sem0, sem1):
    cid = lax.axis_index("c")
    sid = lax.axis_index("s")
    wid = sid * _NCORES + cid
    base = wid * _GPW

    def issue(gg, rowbuf, idxbuf, sem):
        ggc = jnp.minimum(gg, _NG - 1)
        step = ggc >> 7
        r0 = (ggc & 127) * _LANES
        for cb in range(_NCB):
            pltpu.async_copy(
                s_hbm.at[pl.ds((step * _NCB + cb) * (_NR * 128) + r0 * 128,
                               _CHUNK)],
                rowbuf.at[pl.ds(cb * _CHUNK, _CHUNK)],
                sem)
        pltpu.async_copy(idx_hbm.at[pl.ds(ggc * _IDXW, _IDXW)], idxbuf, sem)

    def drain(rowbuf, idxbuf, sem):
        for cb in range(_NCB):
            pltpu.make_async_copy(
                s_hbm.at[pl.ds(0, _CHUNK)],
                rowbuf.at[pl.ds(cb * _CHUNK, _CHUNK)],
                sem).wait()
        pltpu.make_async_copy(idx_hbm.at[pl.ds(0, _IDXW)], idxbuf,
                              sem).wait()

    def compute(gg, rowbuf, idxbuf):
        v = plsc.load_gather(rowbuf, [idxbuf[pl.ds(0, _LANES)]])
        vpos[:] = v
        acc = jnp.exp(v * _INVTEMP)
        for k in range(1, _KTOT):
            vk = plsc.load_gather(rowbuf,
                                  [idxbuf[pl.ds(k * _LANES, _LANES)]])
            acc = acc + jnp.exp(vk * _INVTEMP)
        vsum[:] = acc
        grow = gg * _LANES
        pltpu.sync_copy(vsum, sum_hbm.at[pl.ds(grow, _LANES)])
        pltpu.sync_copy(vpos, pos_hbm.at[pl.ds(grow, _LANES)])

    bufs = ((rowbuf0, idxbuf0, sem0), (rowbuf1, idxbuf1, sem1))
    issue(base, *bufs[0])

    def body(j, carry):
        for par in range(2):
            gg = base + 2 * j + par
            rb, ib, sem = bufs[par]
            nrb, nib, nsem = bufs[1 - par]
            drain(rb, ib, sem)
            issue(gg + 1, nrb, nib, nsem)
            compute(gg, rb, ib)
        return carry

    lax.fori_loop(0, _GPW // 2, body, 0)
    drain(*bufs[0])


@functools.cache
def _get_sc_gather():
    return pl.kernel(
        _sc_gather_body,
        mesh=plsc.VectorSubcoreMesh(core_axis_name="c", subcore_axis_name="s"),
        compiler_params=pltpu.CompilerParams(needs_layout_passes=False),
        out_type=[jax.ShapeDtypeStruct((_NROWS,), jnp.float32),
                  jax.ShapeDtypeStruct((_NROWS,), jnp.float32)],
        scratch_types=[
            pltpu.VMEM((_LANES * _NR,), jnp.float32),
            pltpu.VMEM((_LANES * _NR,), jnp.float32),
            pltpu.VMEM((_KPAD * _LANES,), jnp.int32),
            pltpu.VMEM((_KPAD * _LANES,), jnp.int32),
            pltpu.VMEM((_LANES,), jnp.float32),
            pltpu.VMEM((_LANES,), jnp.float32),
            pltpu.SemaphoreType.DMA,
            pltpu.SemaphoreType.DMA,
        ],
    )


def _loss_body(sum_ref, pos_ref, out_ref):
    se = sum_ref[...]
    ps = pos_ref[...]
    srow = lax.broadcasted_iota(jnp.int32, (_P, _NR), 0)
    rcol = lax.broadcasted_iota(jnp.int32, (_P, _NR), 1)
    t2 = (_T - 1) - srow
    valid = (rcol % _T) < t2
    wgt = jnp.where(valid, 1.0, 0.0) / (_P * _B * t2.astype(jnp.float32))
    out_ref[...] = jnp.sum((jnp.log(se) - ps * _INVTEMP) * wgt).reshape(1, 1)


_loss = pl.pallas_call(
    _loss_body,
    out_shape=jax.ShapeDtypeStruct((1, 1), jnp.float32),
)


def kernel(context, targets, W, b):
    ctx2 = context.reshape(_NR, _D)
    tgt2 = targets.reshape(_NR, _D)
    b3 = b.reshape(_P, 1, _D)
    idx = jnp.asarray(_IDX_CONST)
    sim = _sim(ctx2, tgt2, W, b3)
    se, ps = _get_sc_gather()(sim.reshape(-1), idx)
    return _loss(se.reshape(_P, _NR), ps.reshape(_P, _NR)).reshape(1)
